# baseline (speedup 1.0000x reference)
import numpy as np
import jax
import jax.numpy as jnp
from jax.experimental import pallas as pl
from jax.experimental.pallas import tpu as pltpu

_NB = 256
_S = 16
_BS = 128
_K = 26
_TILE = 256


def _gmv_body(x_ref, wg_ref, bg_ref, w_ref, b_ref, e16_ref, y_ref):
    x = x_ref[...]
    logits = jnp.dot(x, wg_ref[...],
                     preferred_element_type=jnp.float32) + bg_ref[...]

    t = logits.shape[0]
    neg_inf = jnp.float32(-jnp.inf)

    def step(_, carry):
        rem, cnt, thr = carry
        active = cnt < _K
        m = jnp.max(rem, axis=-1, keepdims=True)
        ties = rem >= m
        nt = jnp.sum(ties.astype(jnp.int32), axis=-1, keepdims=True)
        cnt = jnp.where(active, cnt + nt, cnt)
        thr = jnp.where(active, m, thr)
        rem = jnp.where(active & ties, neg_inf, rem)
        return rem, cnt, thr

    _, _, thr = jax.lax.fori_loop(
        0, _K, step,
        (logits, jnp.zeros((t, 1), jnp.int32), jnp.full((t, 1), neg_inf)))

    comp = (logits >= thr).astype(jnp.float32)
    res = comp * logits
    g = res * (jnp.float32(_NB) / jnp.sum(res, axis=-1, keepdims=True))

    e16 = e16_ref[...]
    for j in range(_S):
        gj = g[:, j * _S:(j + 1) * _S]
        gexp = jnp.dot(gj, e16, preferred_element_type=jnp.float32)
        xg = (x * gexp).astype(jnp.bfloat16)
        yj = jnp.dot(xg, w_ref[:, j * _BS:(j + 1) * _BS],
                     preferred_element_type=jnp.float32)
        y_ref[:, j * _BS:(j + 1) * _BS] = yj + b_ref[:, j * _BS:(j + 1) * _BS]


def kernel(x, W_gate, b_gate, weight, bias):
    b_size, d_in = x.shape
    d_out = weight.shape[1]
    nb = W_gate.shape[1]

    perm = (np.arange(nb) % _S) * _S + np.arange(nb) // _S
    wg_p = W_gate[:, perm]
    bg_p = b_gate[perm].reshape(1, nb)
    bias2 = bias.reshape(1, d_out)
    w_bf = weight.astype(jnp.bfloat16)
    e16 = jnp.repeat(jnp.eye(_S, dtype=jnp.float32), _BS, axis=1)

    grid = (b_size // _TILE,)
    y = pl.pallas_call(
        _gmv_body,
        grid=grid,
        in_specs=[
            pl.BlockSpec((_TILE, d_in), lambda i: (i, 0)),
            pl.BlockSpec((d_in, nb), lambda i: (0, 0)),
            pl.BlockSpec((1, nb), lambda i: (0, 0)),
            pl.BlockSpec((d_in, d_out), lambda i: (0, 0)),
            pl.BlockSpec((1, d_out), lambda i: (0, 0)),
            pl.BlockSpec((_S, d_in), lambda i: (0, 0)),
        ],
        out_specs=pl.BlockSpec((_TILE, d_out), lambda i: (i, 0)),
        out_shape=jax.ShapeDtypeStruct((b_size, d_out), jnp.float32),
    )(x, wg_p, bg_p, w_bf, bias2, e16)
    return y


# speedup vs baseline: 1.2782x; 1.2782x over previous
import numpy as np
import jax
import jax.numpy as jnp
from jax.experimental import pallas as pl
from jax.experimental.pallas import tpu as pltpu

_NB = 256
_S = 16
_BS = 128
_K = 26
_TILE = 256


def _gmv_body(x_ref, wg_ref, bg_ref, w_ref, b_ref, e16_ref, y_ref):
    x = x_ref[...]
    logits = jnp.dot(x, wg_ref[...],
                     preferred_element_type=jnp.float32) + bg_ref[...]

    t = logits.shape[0]
    neg_inf = jnp.float32(-jnp.inf)

    def step(_, carry):
        rem, cnt, thr = carry
        active = cnt < _K
        m = jnp.max(rem, axis=-1, keepdims=True)
        ties = rem >= m
        nt = jnp.sum(ties.astype(jnp.int32), axis=-1, keepdims=True)
        cnt = jnp.where(active, cnt + nt, cnt)
        thr = jnp.where(active, m, thr)
        rem = jnp.where(active & ties, neg_inf, rem)
        return rem, cnt, thr

    _, _, thr = jax.lax.fori_loop(
        0, _K, step,
        (logits, jnp.zeros((t, 1), jnp.int32), jnp.full((t, 1), neg_inf)))

    comp = (logits >= thr).astype(jnp.float32)
    res = comp * logits
    g = res * (jnp.float32(_NB) / jnp.sum(res, axis=-1, keepdims=True))

    x_bf = x.astype(jnp.bfloat16)
    g_bf = g.astype(jnp.bfloat16)
    for j in range(_S):
        xg = jnp.concatenate(
            [x_bf[:, i * _BS:(i + 1) * _BS]
             * jnp.broadcast_to(g_bf[:, j * _S + i:j * _S + i + 1], (t, _BS))
             for i in range(_S)], axis=1)
        yj = jnp.dot(xg, w_ref[:, j * _BS:(j + 1) * _BS],
                     preferred_element_type=jnp.float32)
        y_ref[:, j * _BS:(j + 1) * _BS] = yj + b_ref[:, j * _BS:(j + 1) * _BS]


def kernel(x, W_gate, b_gate, weight, bias):
    b_size, d_in = x.shape
    d_out = weight.shape[1]
    nb = W_gate.shape[1]

    perm = (np.arange(nb) % _S) * _S + np.arange(nb) // _S
    wg_p = W_gate[:, perm]
    bg_p = b_gate[perm].reshape(1, nb)
    bias2 = bias.reshape(1, d_out)
    w_bf = weight.astype(jnp.bfloat16)
    e16 = jnp.repeat(jnp.eye(_S, dtype=jnp.bfloat16), _BS, axis=1)

    grid = (b_size // _TILE,)
    y = pl.pallas_call(
        _gmv_body,
        grid=grid,
        in_specs=[
            pl.BlockSpec((_TILE, d_in), lambda i: (i, 0)),
            pl.BlockSpec((d_in, nb), lambda i: (0, 0)),
            pl.BlockSpec((1, nb), lambda i: (0, 0)),
            pl.BlockSpec((d_in, d_out), lambda i: (0, 0)),
            pl.BlockSpec((1, d_out), lambda i: (0, 0)),
            pl.BlockSpec((_S, d_in), lambda i: (0, 0)),
        ],
        out_specs=pl.BlockSpec((_TILE, d_out), lambda i: (i, 0)),
        out_shape=jax.ShapeDtypeStruct((b_size, d_out), jnp.float32),
    )(x, wg_p, bg_p, w_bf, bias2, e16)
    return y


# speedup vs baseline: 2.0966x; 1.6403x over previous
import numpy as np
import jax
import jax.numpy as jnp
from jax.experimental import pallas as pl
from jax.experimental.pallas import tpu as pltpu

_NB = 256
_S = 16
_BS = 128
_K = 26
_TILE = 256


def _gmv_body(x_ref, wg_ref, bg_ref, w_ref, b_ref, e16_ref, y_ref):
    x = x_ref[...]
    logits = jnp.dot(x, wg_ref[...],
                     preferred_element_type=jnp.float32) + bg_ref[...]

    t = logits.shape[0]
    neg_inf = jnp.float32(-jnp.inf)

    rem = logits
    cnt = jnp.zeros((t, 1), jnp.float32)
    thr = jnp.full((t, 1), neg_inf)
    for _ in range(_K):
        m = jnp.max(rem, axis=-1, keepdims=True)
        ties = rem >= m
        nt = jnp.sum(jnp.where(ties, 1.0, 0.0), axis=-1, keepdims=True)
        thr = jnp.where(cnt < _K, m, thr)
        cnt = cnt + nt
        rem = jnp.where(ties, neg_inf, rem)

    comp = (logits >= thr).astype(jnp.float32)
    res = comp * logits
    g = res * (jnp.float32(_NB) / jnp.sum(res, axis=-1, keepdims=True))

    x_bf = x.astype(jnp.bfloat16)
    g_bf = g.astype(jnp.bfloat16)
    for j in range(_S):
        xg = jnp.concatenate(
            [x_bf[:, i * _BS:(i + 1) * _BS]
             * jnp.broadcast_to(g_bf[:, j * _S + i:j * _S + i + 1], (t, _BS))
             for i in range(_S)], axis=1)
        yj = jnp.dot(xg, w_ref[:, j * _BS:(j + 1) * _BS],
                     preferred_element_type=jnp.float32)
        y_ref[:, j * _BS:(j + 1) * _BS] = yj + b_ref[:, j * _BS:(j + 1) * _BS]


def kernel(x, W_gate, b_gate, weight, bias):
    b_size, d_in = x.shape
    d_out = weight.shape[1]
    nb = W_gate.shape[1]

    perm = (np.arange(nb) % _S) * _S + np.arange(nb) // _S
    wg_p = W_gate[:, perm]
    bg_p = b_gate[perm].reshape(1, nb)
    bias2 = bias.reshape(1, d_out)
    w_bf = weight.astype(jnp.bfloat16)
    e16 = jnp.repeat(jnp.eye(_S, dtype=jnp.bfloat16), _BS, axis=1)

    grid = (b_size // _TILE,)
    y = pl.pallas_call(
        _gmv_body,
        grid=grid,
        in_specs=[
            pl.BlockSpec((_TILE, d_in), lambda i: (i, 0)),
            pl.BlockSpec((d_in, nb), lambda i: (0, 0)),
            pl.BlockSpec((1, nb), lambda i: (0, 0)),
            pl.BlockSpec((d_in, d_out), lambda i: (0, 0)),
            pl.BlockSpec((1, d_out), lambda i: (0, 0)),
            pl.BlockSpec((_S, d_in), lambda i: (0, 0)),
        ],
        out_specs=pl.BlockSpec((_TILE, d_out), lambda i: (i, 0)),
        out_shape=jax.ShapeDtypeStruct((b_size, d_out), jnp.float32),
    )(x, wg_p, bg_p, w_bf, bias2, e16)
    return y


# speedup vs baseline: 2.3327x; 1.1126x over previous
import numpy as np
import jax
import jax.numpy as jnp
from jax.experimental import pallas as pl
from jax.experimental.pallas import tpu as pltpu

_NB = 256
_S = 16
_BS = 128
_K = 26
_TILE = 512


def _gmv_body(x_ref, wg_ref, bg_ref, w_ref, b_ref, y_ref):
    x = x_ref[...]
    logits = jnp.dot(x, wg_ref[...],
                     preferred_element_type=jnp.float32) + bg_ref[...]

    t = logits.shape[0]
    neg_inf = jnp.float32(-jnp.inf)

    rem = logits
    cnt = jnp.zeros((t, 1), jnp.float32)
    thr = jnp.full((t, 1), neg_inf)
    for _ in range(_K):
        m = jnp.max(rem, axis=-1, keepdims=True)
        ties = rem >= m
        nt = jnp.sum(jnp.where(ties, 1.0, 0.0), axis=-1, keepdims=True)
        thr = jnp.where(cnt < _K, m, thr)
        cnt = cnt + nt
        rem = jnp.where(ties, neg_inf, rem)

    comp = (logits >= thr).astype(jnp.float32)
    res = comp * logits
    g = res * (jnp.float32(_NB) / jnp.sum(res, axis=-1, keepdims=True))

    x_bf = x.astype(jnp.bfloat16)
    g_bf = g.astype(jnp.bfloat16)
    for j in range(_S):
        xg = jnp.concatenate(
            [x_bf[:, i * _BS:(i + 1) * _BS]
             * jnp.broadcast_to(g_bf[:, j * _S + i:j * _S + i + 1], (t, _BS))
             for i in range(_S)], axis=1)
        yj = jnp.dot(xg, w_ref[:, j * _BS:(j + 1) * _BS],
                     preferred_element_type=jnp.float32)
        y_ref[:, j * _BS:(j + 1) * _BS] = yj + b_ref[:, j * _BS:(j + 1) * _BS]


def kernel(x, W_gate, b_gate, weight, bias):
    b_size, d_in = x.shape
    d_out = weight.shape[1]
    nb = W_gate.shape[1]

    perm = (np.arange(nb) % _S) * _S + np.arange(nb) // _S
    wg_p = W_gate[:, perm]
    bg_p = b_gate[perm].reshape(1, nb)
    bias2 = bias.reshape(1, d_out)
    w_bf = weight.astype(jnp.bfloat16)

    grid = (b_size // _TILE,)
    y = pl.pallas_call(
        _gmv_body,
        grid=grid,
        in_specs=[
            pl.BlockSpec((_TILE, d_in), lambda i: (i, 0)),
            pl.BlockSpec((d_in, nb), lambda i: (0, 0)),
            pl.BlockSpec((1, nb), lambda i: (0, 0)),
            pl.BlockSpec((d_in, d_out), lambda i: (0, 0)),
            pl.BlockSpec((1, d_out), lambda i: (0, 0)),
        ],
        out_specs=pl.BlockSpec((_TILE, d_out), lambda i: (i, 0)),
        out_shape=jax.ShapeDtypeStruct((b_size, d_out), jnp.float32),
    )(x, wg_p, bg_p, w_bf, bias2)
    return y
